# baseline (speedup 1.0000x reference)
import functools

import jax
import jax.numpy as jnp
from jax import lax
from jax.experimental import pallas as pl
from jax.experimental.pallas import tpu as pltpu
from jax.experimental.pallas import tpu_sc as plsc

_NC, _NS = 2, 16
_NW = _NC * _NS
_L = 200
_EMB = 64
_CA, _CB = 104, 96


def _sc_pool_body(text_hbm, embed_hbm, out_hbm, idx_v, rows_v, out_v, sem_a, sem_b):
    wid = lax.axis_index("s") * _NC + lax.axis_index("c")
    bpw = out_v.shape[0]
    base = wid * bpw

    def elem(i, carry):
        pltpu.sync_copy(text_hbm.at[base + i], idx_v)
        cp_a = pltpu.async_copy(
            embed_hbm.at[idx_v.at[pl.ds(0, _CA)]], rows_v.at[pl.ds(0, _CA)], sem_a)
        cp_b = pltpu.async_copy(
            embed_hbm.at[idx_v.at[pl.ds(_CA, _CB)]], rows_v.at[pl.ds(_CA, _CB)], sem_b)
        cp_a.wait()
        cp_b.wait()

        def rows8(j, accs):
            a = list(accs)
            for u in range(8):
                r = j * 8 + u
                for c in range(4):
                    a[c] = a[c] + rows_v[r, pl.ds(16 * c, 16)]
            return tuple(a)

        zero = jnp.zeros((16,), jnp.float32)
        accs = lax.fori_loop(0, _L // 8, rows8, (zero, zero, zero, zero))
        scale = jnp.float32(1.0 / _L)
        for c in range(4):
            out_v[i, pl.ds(16 * c, 16)] = accs[c] * scale
        return carry

    lax.fori_loop(0, bpw, elem, 0)
    pltpu.sync_copy(out_v, out_hbm.at[pl.ds(base, bpw)])


@functools.cache
def _make_sc_pool(batch):
    bpw = batch // _NW
    mesh = plsc.VectorSubcoreMesh(core_axis_name="c", subcore_axis_name="s")
    return pl.kernel(
        _sc_pool_body,
        out_type=jax.ShapeDtypeStruct((batch, _EMB), jnp.float32),
        mesh=mesh,
        scratch_types=[
            pltpu.VMEM((_L,), jnp.int32),
            pltpu.VMEM((_L, _EMB), jnp.float32),
            pltpu.VMEM((bpw, _EMB), jnp.float32),
            pltpu.SemaphoreType.DMA,
            pltpu.SemaphoreType.DMA,
        ],
        compiler_params=pltpu.CompilerParams(use_tc_tiling_on_sc=False),
    )


def _mlp_body(pooled_ref, w1_ref, b1_ref, w2_ref, b2_ref, out_ref):
    h = jnp.tanh(
        jnp.dot(pooled_ref[...], w1_ref[...], preferred_element_type=jnp.float32)
        + b1_ref[...][None, :])
    out_ref[...] = (
        jnp.dot(h, w2_ref[...], preferred_element_type=jnp.float32)
        + b2_ref[...][None, :])


def kernel(text, embed, w1, b1, w2, b2):
    batch = text.shape[0]
    pooled = _make_sc_pool(batch)(text, embed)
    return pl.pallas_call(
        _mlp_body,
        out_shape=jax.ShapeDtypeStruct((batch, w2.shape[1]), jnp.float32),
    )(pooled, w1, b1, w2, b2)
